# Initial kernel scaffold; baseline (speedup 1.0000x reference)
import jax
import jax.numpy as jnp
from jax.experimental import pallas as pl


def kernel(x, edge_index, edge_attr, Ws, bs, Wr, br, We, be):
    raise NotImplementedError("write your pallas kernel here")


# speedup vs baseline: 3.5928x; 3.5928x over previous
import functools

import jax
import jax.numpy as jnp
from jax import lax
from jax.experimental import pallas as pl
from jax.experimental.pallas import tpu as pltpu
from jax.experimental.pallas import tpu_sc as plsc

_NC = 2
_NS = 16
_NW = _NC * _NS
_LANES = 16


def _node_mm_body(x_ref, ws_ref, wr_ref, os_ref, or_ref):
    xv = x_ref[...]
    os_ref[...] = jnp.dot(xv, ws_ref[...], preferred_element_type=jnp.float32)
    or_ref[...] = jnp.dot(xv, wr_ref[...], preferred_element_type=jnp.float32)


def _node_transform(x, wst, wrt):
    n, d = x.shape
    bn = 2000
    return pl.pallas_call(
        _node_mm_body,
        grid=(n // bn,),
        in_specs=[
            pl.BlockSpec((bn, d), lambda i: (i, 0)),
            pl.BlockSpec((d, d), lambda i: (0, 0)),
            pl.BlockSpec((d, d), lambda i: (0, 0)),
        ],
        out_specs=[
            pl.BlockSpec((bn, d), lambda i: (i, 0)),
            pl.BlockSpec((bn, d), lambda i: (i, 0)),
        ],
        out_shape=[
            jax.ShapeDtypeStruct((n, d), jnp.float32),
            jax.ShapeDtypeStruct((n, d), jnp.float32),
        ],
    )(x, wst, wrt)


def _make_gather_sum(e, d, c):
    epw = e // _NW
    mesh = plsc.VectorSubcoreMesh(core_axis_name="c", subcore_axis_name="s")

    @functools.partial(
        pl.kernel,
        mesh=mesh,
        out_type=jax.ShapeDtypeStruct((e, d), jnp.float32),
        scratch_types=[
            pltpu.VMEM((epw,), jnp.int32),
            pltpu.VMEM((epw,), jnp.int32),
            pltpu.VMEM((c, d), jnp.float32),
            pltpu.VMEM((c, d), jnp.float32),
            pltpu.SemaphoreType.DMA,
        ],
    )
    def gather_sum(xs_hbm, xr_hbm, si_hbm, ri_hbm, out_hbm,
                   si_v, ri_v, a_v, b_v, sem):
        wid = lax.axis_index("s") * _NC + lax.axis_index("c")
        base = wid * epw
        pltpu.sync_copy(si_hbm.at[pl.ds(base, epw)], si_v)
        pltpu.sync_copy(ri_hbm.at[pl.ds(base, epw)], ri_v)

        @pl.loop(0, epw, step=c)
        def _chunk(off):
            cp_a = pltpu.async_copy(xs_hbm.at[si_v.at[pl.ds(off, c)]], a_v, sem)
            cp_b = pltpu.async_copy(xr_hbm.at[ri_v.at[pl.ds(off, c)]], b_v, sem)
            cp_a.wait()
            cp_b.wait()

            @pl.loop(0, c)
            def _row(i):
                for j in range(d // _LANES):
                    slc = (i, pl.ds(j * _LANES, _LANES))
                    a_v[slc] = a_v[slc] + b_v[slc]

            pltpu.sync_copy(a_v, out_hbm.at[pl.ds(base + off, c)])

    return gather_sum


def _edge_mm_body(g_ref, ea_ref, we_ref, b_ref, o_ref):
    o_ref[...] = (
        g_ref[...]
        + jnp.dot(ea_ref[...], we_ref[...], preferred_element_type=jnp.float32)
        + b_ref[...]
    )


def _edge_combine(g, edge_attr, wet, btot):
    e, d = g.shape
    de = edge_attr.shape[1]
    be = 3200
    return pl.pallas_call(
        _edge_mm_body,
        grid=(e // be,),
        in_specs=[
            pl.BlockSpec((be, d), lambda i: (i, 0)),
            pl.BlockSpec((be, de), lambda i: (i, 0)),
            pl.BlockSpec((de, d), lambda i: (0, 0)),
            pl.BlockSpec((1, d), lambda i: (0, 0)),
        ],
        out_specs=pl.BlockSpec((be, d), lambda i: (i, 0)),
        out_shape=jax.ShapeDtypeStruct((e, d), jnp.float32),
    )(g, edge_attr, wet, btot)


def kernel(x, edge_index, edge_attr, Ws, bs, Wr, br, We, be):
    e = edge_index.shape[1]
    d = x.shape[1]
    senders = edge_index[0]
    receivers = edge_index[1]
    xs, xr = _node_transform(x, Ws.T, Wr.T)
    gsum = _make_gather_sum(e, d, 200)(xs, xr, senders, receivers)
    btot = (bs + br + be).reshape(1, d)
    return _edge_combine(gsum, edge_attr, We.T, btot)


# speedup vs baseline: 4.1589x; 1.1576x over previous
import functools

import jax
import jax.numpy as jnp
from jax import lax
from jax.experimental import pallas as pl
from jax.experimental.pallas import tpu as pltpu
from jax.experimental.pallas import tpu_sc as plsc

_NC = 2
_NS = 16
_NW = _NC * _NS
_LANES = 16


def _node_mm_body(x_ref, ws_ref, wr_ref, os_ref, or_ref):
    xv = x_ref[...]
    os_ref[...] = jnp.dot(xv, ws_ref[...], preferred_element_type=jnp.float32)
    or_ref[...] = jnp.dot(xv, wr_ref[...], preferred_element_type=jnp.float32)


def _node_transform(x, wst, wrt):
    n, d = x.shape
    bn = 2000
    return pl.pallas_call(
        _node_mm_body,
        grid=(n // bn,),
        in_specs=[
            pl.BlockSpec((bn, d), lambda i: (i, 0)),
            pl.BlockSpec((d, d), lambda i: (0, 0)),
            pl.BlockSpec((d, d), lambda i: (0, 0)),
        ],
        out_specs=[
            pl.BlockSpec((bn, d), lambda i: (i, 0)),
            pl.BlockSpec((bn, d), lambda i: (i, 0)),
        ],
        out_shape=[
            jax.ShapeDtypeStruct((n, d), jnp.float32),
            jax.ShapeDtypeStruct((n, d), jnp.float32),
        ],
    )(x, wst, wrt)


def _make_gather_sum(e, d, c):
    epw = e // _NW
    nch = epw // c
    assert epw % c == 0 and nch % 2 == 0 and c % 8 == 0
    mesh = plsc.VectorSubcoreMesh(core_axis_name="c", subcore_axis_name="s")

    @functools.partial(
        pl.kernel,
        mesh=mesh,
        out_type=jax.ShapeDtypeStruct((e, d), jnp.float32),
        scratch_types=[
            pltpu.VMEM((epw,), jnp.int32),
            pltpu.VMEM((epw,), jnp.int32),
            pltpu.VMEM((c, d), jnp.float32),
            pltpu.VMEM((c, d), jnp.float32),
            pltpu.VMEM((c, d), jnp.float32),
            pltpu.VMEM((c, d), jnp.float32),
            pltpu.SemaphoreType.DMA,
            pltpu.SemaphoreType.DMA,
            pltpu.SemaphoreType.DMA,
            pltpu.SemaphoreType.DMA,
            pltpu.SemaphoreType.DMA,
            pltpu.SemaphoreType.DMA,
        ],
    )
    def gather_sum(xs_hbm, xr_hbm, si_hbm, ri_hbm, out_hbm,
                   si_v, ri_v, a0, b0, a1, b1,
                   ga0, gb0, ga1, gb1, os0, os1):
        wid = lax.axis_index("s") * _NC + lax.axis_index("c")
        base = wid * epw
        abuf, bbuf = (a0, a1), (b0, b1)
        asem, bsem = (ga0, ga1), (gb0, gb1)
        osem = (os0, os1)
        pltpu.sync_copy(si_hbm.at[pl.ds(base, epw)], si_v)
        pltpu.sync_copy(ri_hbm.at[pl.ds(base, epw)], ri_v)

        def start_gather(off, p):
            pltpu.async_copy(xs_hbm.at[si_v.at[pl.ds(off, c)]], abuf[p], asem[p])
            pltpu.async_copy(xr_hbm.at[ri_v.at[pl.ds(off, c)]], bbuf[p], bsem[p])

        def wait_gather(p):
            pltpu.make_async_copy(xs_hbm.at[pl.ds(0, c)], abuf[p], asem[p]).wait()
            pltpu.make_async_copy(xr_hbm.at[pl.ds(0, c)], bbuf[p], bsem[p]).wait()

        start_gather(0, 0)
        start_gather(c, 1)

        @pl.loop(0, epw, step=2 * c)
        def _pair(off):
            for p in range(2):
                o = off + p * c
                wait_gather(p)

                @pl.loop(0, c)
                def _row(i):
                    for j in range(d // _LANES):
                        slc = (i, pl.ds(j * _LANES, _LANES))
                        abuf[p][slc] = abuf[p][slc] + bbuf[p][slc]

                pltpu.async_copy(abuf[p], out_hbm.at[pl.ds(base + o, c)],
                                 osem[p])

                @pl.when(o + 2 * c < epw)
                def _prefetch():
                    pltpu.make_async_copy(
                        abuf[p], out_hbm.at[pl.ds(base, c)], osem[p]).wait()
                    start_gather(o + 2 * c, p)

        pltpu.make_async_copy(a0, out_hbm.at[pl.ds(base, c)], os0).wait()
        pltpu.make_async_copy(a1, out_hbm.at[pl.ds(base, c)], os1).wait()

    return gather_sum


def _edge_mm_body(g_ref, ea_ref, we_ref, b_ref, o_ref):
    o_ref[...] = (
        g_ref[...]
        + jnp.dot(ea_ref[...], we_ref[...], preferred_element_type=jnp.float32)
        + b_ref[...]
    )


def _edge_combine(g, edge_attr, wet, btot):
    e, d = g.shape
    de = edge_attr.shape[1]
    be = 3200
    return pl.pallas_call(
        _edge_mm_body,
        grid=(e // be,),
        in_specs=[
            pl.BlockSpec((be, d), lambda i: (i, 0)),
            pl.BlockSpec((be, de), lambda i: (i, 0)),
            pl.BlockSpec((de, d), lambda i: (0, 0)),
            pl.BlockSpec((1, d), lambda i: (0, 0)),
        ],
        out_specs=pl.BlockSpec((be, d), lambda i: (i, 0)),
        out_shape=jax.ShapeDtypeStruct((e, d), jnp.float32),
    )(g, edge_attr, wet, btot)


def kernel(x, edge_index, edge_attr, Ws, bs, Wr, br, We, be):
    e = edge_index.shape[1]
    d = x.shape[1]
    senders = edge_index[0]
    receivers = edge_index[1]
    xs, xr = _node_transform(x, Ws.T, Wr.T)
    gsum = _make_gather_sum(e, d, 200)(xs, xr, senders, receivers)
    btot = (bs + br + be).reshape(1, d)
    return _edge_combine(gsum, edge_attr, We.T, btot)
